# Initial kernel scaffold; baseline (speedup 1.0000x reference)
import jax
import jax.numpy as jnp
from jax.experimental import pallas as pl


def kernel(emb, posemb, segemb, gamma, beta, x, seg, pos):
    raise NotImplementedError("write your pallas kernel here")


# speedup vs baseline: 1.7893x; 1.7893x over previous
import functools

import jax
import jax.numpy as jnp
from jax import lax
from jax.experimental import pallas as pl
from jax.experimental.pallas import tpu as pltpu
from jax.experimental.pallas import tpu_sc as plsc

VOC = 1000000
DIM = 64
MAXLEN = 512
SEGN = 2
B = 1024
L = 200
EPS = 1e-06

N = B * L
NC = 2
NS = 16
NW = NC * NS
PER_W = N // NW
C = 256
NCHUNK = PER_W // C
KSUB = C // 128
NV = DIM // 16


_GDIMS = lax.GatherDimensionNumbers(
    offset_dims=(), collapsed_slice_dims=(0,), start_index_map=(0,))


def _allsum(v, perms):
    for p in perms:
        v = v + lax.gather(v, p, _GDIMS, (1,),
                           mode=lax.GatherScatterMode.PROMISE_IN_BOUNDS)
    return v


def _rsqrt(v):
    i = lax.bitcast_convert_type(v, jnp.int32)
    i = jnp.int32(0x5F3759DF) - lax.shift_right_logical(i, 1)
    y = lax.bitcast_convert_type(i, jnp.float32)
    for _ in range(3):
        y = y * (1.5 - 0.5 * v * y * y)
    return y


def _body(emb_h, pose_h, sege_h, gam_h, bet_h, x_h, s_h, p_h, out_h,
          xidx_v, pidx_v, sidx_v, rows_e, rows_p, rows_o,
          gam_v, bet_v, seg_v, sem):
    cid = lax.axis_index("c")
    sid = lax.axis_index("s")
    wid = sid * NC + cid

    pltpu.sync_copy(gam_h, gam_v)
    pltpu.sync_copy(bet_h, bet_v)
    pltpu.sync_copy(sege_h, seg_v)

    g = [gam_v[pl.ds(16 * k, 16)] for k in range(NV)]
    b = [bet_v[pl.ds(16 * k, 16)] for k in range(NV)]
    s0 = [seg_v[0, pl.ds(16 * k, 16)] for k in range(NV)]
    sd = [seg_v[1, pl.ds(16 * k, 16)] - s0[k] for k in range(NV)]

    lanes = lax.iota(jnp.int32, 16)
    perms = [(lanes ^ (1 << k))[:, None] for k in range(4)]

    def group_body(gi, carry):
        sff = sidx_v[pl.ds(gi * 16, 16)].astype(jnp.float32)
        for j in range(16):
            i = gi * 16 + j
            sf = sff[j]
            hs = []
            for k in range(NV):
                e = rows_e[i, pl.ds(16 * k, 16)]
                p = rows_p[i, pl.ds(16 * k, 16)]
                hs.append(e + p + s0[k] + sf * sd[k])
            sv = (hs[0] + hs[1]) + (hs[2] + hs[3])
            qv = (hs[0] * hs[0] + hs[1] * hs[1]) + (hs[2] * hs[2] + hs[3] * hs[3])
            mu = _allsum(sv, perms) * (1.0 / DIM)
            var = _allsum(qv, perms) * (1.0 / DIM) - mu * mu
            rv = _rsqrt(var + EPS)
            for k in range(NV):
                rows_o[i, pl.ds(16 * k, 16)] = (hs[k] - mu) * rv * g[k] + b[k]
        return carry

    def chunk_body(c, carry):
        base = wid * PER_W + c * C
        rb = wid * (PER_W // 128) + c * KSUB
        pltpu.sync_copy(x_h.at[pl.ds(rb, KSUB)], xidx_v)
        pltpu.sync_copy(p_h.at[pl.ds(rb, KSUB)], pidx_v)
        pltpu.sync_copy(s_h.at[pl.ds(base, C)], sidx_v)
        for k in range(KSUB):
            pltpu.async_copy(emb_h.at[xidx_v.at[k]],
                             rows_e.at[pl.ds(k * 128, 128)], sem).wait()
            pltpu.async_copy(pose_h.at[pidx_v.at[k]],
                             rows_p.at[pl.ds(k * 128, 128)], sem).wait()
        lax.fori_loop(0, C // 16, group_body, 0, unroll=False)
        pltpu.sync_copy(rows_o, out_h.at[pl.ds(base, C)])
        return carry

    lax.fori_loop(0, NCHUNK, chunk_body, 0, unroll=False)


_emb_ln = functools.partial(
    pl.kernel,
    out_type=jax.ShapeDtypeStruct((N, DIM), jnp.float32),
    mesh=plsc.VectorSubcoreMesh(core_axis_name="c", subcore_axis_name="s"),
    compiler_params=pltpu.CompilerParams(use_tc_tiling_on_sc=False),
    scratch_types=[
        pltpu.VMEM((KSUB, 128), jnp.int32),
        pltpu.VMEM((KSUB, 128), jnp.int32),
        pltpu.VMEM((C,), jnp.int32),
        pltpu.VMEM((C, DIM), jnp.float32),
        pltpu.VMEM((C, DIM), jnp.float32),
        pltpu.VMEM((C, DIM), jnp.float32),
        pltpu.VMEM((DIM,), jnp.float32),
        pltpu.VMEM((DIM,), jnp.float32),
        pltpu.VMEM((SEGN, DIM), jnp.float32),
        pltpu.SemaphoreType.DMA,
    ],
)(_body)


@jax.jit
def kernel(emb, posemb, segemb, gamma, beta, x, seg, pos):
    xf = x.reshape(N // 128, 128)
    pf = pos.reshape(N // 128, 128)
    sf = seg.reshape(N)
    out = _emb_ln(emb, posemb, segemb, gamma, beta, xf, sf, pf)
    return out.reshape(B, L, DIM)


# speedup vs baseline: 1.8481x; 1.0329x over previous
import functools

import jax
import jax.numpy as jnp
from jax import lax
from jax.experimental import pallas as pl
from jax.experimental.pallas import tpu as pltpu
from jax.experimental.pallas import tpu_sc as plsc

VOC = 1000000
DIM = 64
MAXLEN = 512
SEGN = 2
B = 1024
L = 200
EPS = 1e-06

N = B * L
NC = 2
NS = 16
NW = NC * NS
PER_W = N // NW
CR = 128
CNK = PER_W // CR
NPAIR = CNK // 2
GRP = CR // 16
NV = DIM // 16

_GDIMS = lax.GatherDimensionNumbers(
    offset_dims=(), collapsed_slice_dims=(0,), start_index_map=(0,))


def _allsum(v, perms):
    for p in perms:
        v = v + lax.gather(v, p, _GDIMS, (1,),
                           mode=lax.GatherScatterMode.PROMISE_IN_BOUNDS)
    return v


def _rsqrt(v):
    i = lax.bitcast_convert_type(v, jnp.int32)
    i = jnp.int32(0x5F3759DF) - lax.shift_right_logical(i, 1)
    y = lax.bitcast_convert_type(i, jnp.float32)
    for _ in range(3):
        y = y * (1.5 - 0.5 * v * y * y)
    return y


def _body(emb_h, pose_h, sege_h, gam_h, bet_h, x_h, s_h, p_h, out_h,
          xidx0, xidx1, pidx0, pidx1, sidx0, sidx1,
          re0, re1, rp0, rp1, ro0, ro1,
          gam_v, bet_v, seg_v,
          seme0, seme1, semp0, semp1):
    cid = lax.axis_index("c")
    sid = lax.axis_index("s")
    wid = sid * NC + cid

    xidx = [xidx0, xidx1]
    pidx = [pidx0, pidx1]
    sidx = [sidx0, sidx1]
    re = [re0, re1]
    rp = [rp0, rp1]
    ro = [ro0, ro1]
    seme = [seme0, seme1]
    semp = [semp0, semp1]

    pltpu.sync_copy(gam_h, gam_v)
    pltpu.sync_copy(bet_h, bet_v)
    pltpu.sync_copy(sege_h, seg_v)

    g = [gam_v[pl.ds(16 * k, 16)] for k in range(NV)]
    b = [bet_v[pl.ds(16 * k, 16)] for k in range(NV)]
    s0 = [seg_v[0, pl.ds(16 * k, 16)] for k in range(NV)]
    sd = [seg_v[1, pl.ds(16 * k, 16)] - s0[k] for k in range(NV)]

    lanes = lax.iota(jnp.int32, 16)
    perms = [(lanes ^ (1 << k))[:, None] for k in range(4)]

    def issue(c, bs):
        rb = wid * CNK + c
        pltpu.sync_copy(x_h.at[pl.ds(rb, 1)], xidx[bs])
        pltpu.sync_copy(p_h.at[pl.ds(rb, 1)], pidx[bs])
        pltpu.sync_copy(s_h.at[pl.ds(rb * CR, CR)], sidx[bs])
        pltpu.async_copy(emb_h.at[xidx[bs].at[0]], re[bs], seme[bs])
        pltpu.async_copy(pose_h.at[pidx[bs].at[0]], rp[bs], semp[bs])

    def wait_gathers(bs):
        pltpu.make_async_copy(emb_h.at[xidx[bs].at[0]], re[bs],
                              seme[bs]).wait()
        pltpu.make_async_copy(pose_h.at[pidx[bs].at[0]], rp[bs],
                              semp[bs]).wait()

    def compute(bs):
        re_b, rp_b, ro_b, sidx_b = re[bs], rp[bs], ro[bs], sidx[bs]

        @plsc.parallel_loop(0, GRP)
        def _(gi):
            sff = sidx_b[pl.ds(gi * 16, 16)].astype(jnp.float32)
            for j in range(16):
                i = gi * 16 + j
                sf = sff[j]
                hs = []
                for k in range(NV):
                    e = re_b[i, pl.ds(16 * k, 16)]
                    p = rp_b[i, pl.ds(16 * k, 16)]
                    hs.append(e + p + s0[k] + sf * sd[k])
                sv = (hs[0] + hs[1]) + (hs[2] + hs[3])
                qv = (hs[0] * hs[0] + hs[1] * hs[1]) \
                    + (hs[2] * hs[2] + hs[3] * hs[3])
                mu = _allsum(sv, perms) * (1.0 / DIM)
                var = _allsum(qv, perms) * (1.0 / DIM) - mu * mu
                rv = _rsqrt(var + EPS)
                for k in range(NV):
                    ro_b[i, pl.ds(16 * k, 16)] = \
                        (hs[k] - mu) * rv * g[k] + b[k]

    def pair_body(i, carry):
        wait_gathers(0)
        issue(2 * i + 1, 1)
        compute(0)
        pltpu.sync_copy(ro[0], out_h.at[pl.ds((wid * CNK + 2 * i) * CR, CR)])

        wait_gathers(1)

        @pl.when(i < NPAIR - 1)
        def _():
            issue(2 * i + 2, 0)

        compute(1)
        pltpu.sync_copy(ro[1],
                        out_h.at[pl.ds((wid * CNK + 2 * i + 1) * CR, CR)])
        return carry

    issue(0, 0)
    lax.fori_loop(0, NPAIR, pair_body, 0, unroll=False)


_emb_ln = functools.partial(
    pl.kernel,
    out_type=jax.ShapeDtypeStruct((N, DIM), jnp.float32),
    mesh=plsc.VectorSubcoreMesh(core_axis_name="c", subcore_axis_name="s"),
    compiler_params=pltpu.CompilerParams(use_tc_tiling_on_sc=False),
    scratch_types=[
        pltpu.VMEM((1, CR), jnp.int32),
        pltpu.VMEM((1, CR), jnp.int32),
        pltpu.VMEM((1, CR), jnp.int32),
        pltpu.VMEM((1, CR), jnp.int32),
        pltpu.VMEM((CR,), jnp.int32),
        pltpu.VMEM((CR,), jnp.int32),
        pltpu.VMEM((CR, DIM), jnp.float32),
        pltpu.VMEM((CR, DIM), jnp.float32),
        pltpu.VMEM((CR, DIM), jnp.float32),
        pltpu.VMEM((CR, DIM), jnp.float32),
        pltpu.VMEM((CR, DIM), jnp.float32),
        pltpu.VMEM((CR, DIM), jnp.float32),
        pltpu.VMEM((DIM,), jnp.float32),
        pltpu.VMEM((DIM,), jnp.float32),
        pltpu.VMEM((SEGN, DIM), jnp.float32),
        pltpu.SemaphoreType.DMA,
        pltpu.SemaphoreType.DMA,
        pltpu.SemaphoreType.DMA,
        pltpu.SemaphoreType.DMA,
    ],
)(_body)


@jax.jit
def kernel(emb, posemb, segemb, gamma, beta, x, seg, pos):
    xf = x.reshape(N // CR, CR)
    pf = pos.reshape(N // CR, CR)
    sf = seg.reshape(N)
    out = _emb_ln(emb, posemb, segemb, gamma, beta, xf, sf, pf)
    return out.reshape(B, L, DIM)
